# baseline (speedup 1.0000x reference)
import functools
import jax
import jax.numpy as jnp
from jax import lax
from jax.experimental import pallas as pl
from jax.experimental.pallas import tpu as pltpu
from jax.experimental.pallas import tpu_sc as plsc

RX = RY = RZ = 256
HN = 2.0
F = 200000
BLK = 1024
GRID = (F + BLK - 1) // BLK

NC, NS = 2, 16
K = 8192
KROWS = K // 128
NP = 56 * F
NCH = -(-NP // (NS * K))
NCH_TOT = NS * NCH
NP_PAD = NCH_TOT * K

REG_P = 16
PLANE_W = RY * RX
ACC_N = REG_P * PLANE_W + 8
DUMP = REG_P * PLANE_W
SWEEPS = (RZ // NC) // REG_P


def _prep_body(v0r, v1r, v2r, idxo, wo):
    v0 = v0r[...]
    v1 = v1r[...]
    v2 = v2r[...]
    pts = (
        v0,
        v1,
        v2,
        (v0 + v1 + v2) / 3.0,
        (v0 + v1) * 0.5,
        (v1 + v2) * 0.5,
        (v0 + v2) * 0.5,
    )
    res = jnp.float32(RX)
    for p_i, p in enumerate(pts):
        g = (p / HN + 0.5) * res - 0.5
        g0f = jnp.floor(g)
        fr = g - g0f
        g0 = g0f.astype(jnp.int32)
        for dx in (0, 1):
            wx = fr[0:1, :] if dx == 1 else (1.0 - fr[0:1, :])
            ix = jnp.clip(g0[0:1, :] + dx, 0, RX - 1)
            for dy in (0, 1):
                wy = fr[1:2, :] if dy == 1 else (1.0 - fr[1:2, :])
                iy = jnp.clip(g0[1:2, :] + dy, 0, RY - 1)
                for dz in (0, 1):
                    wz = fr[2:3, :] if dz == 1 else (1.0 - fr[2:3, :])
                    iz = jnp.clip(g0[2:3, :] + dz, 0, RZ - 1)
                    row = p_i * 8 + dx * 4 + dy * 2 + dz
                    flat = (iz * RY + iy) * RX + ix
                    idxo[row : row + 1, :] = flat
                    wo[row : row + 1, :] = wx * wy * wz


def _make_pairs(v0t, v1t, v2t):
    return pl.pallas_call(
        _prep_body,
        grid=(GRID,),
        in_specs=[pl.BlockSpec((3, BLK), lambda i: (0, i))] * 3,
        out_specs=[
            pl.BlockSpec((56, BLK), lambda i: (0, i)),
            pl.BlockSpec((56, BLK), lambda i: (0, i)),
        ],
        out_shape=[
            jax.ShapeDtypeStruct((56, F), jnp.int32),
            jax.ShapeDtypeStruct((56, F), jnp.float32),
        ],
    )(v0t, v1t, v2t)


def _sc_scatter_body(idx_hbm, w_hbm, z_hbm, out_hbm, acc, ib, wb, stage):
    c = lax.axis_index("c")
    s = lax.axis_index("s")

    for r in range(SWEEPS):
        lo = c * (RZ // NC) + r * REG_P

        @pl.when(s == 0)
        def _():
            pltpu.sync_copy(z_hbm, acc)

        plsc.subcore_barrier()

        @pl.loop(0, NCH)
        def _chunk(ci):
            row = s * NCH + ci
            pltpu.sync_copy(idx_hbm.at[row], ib)
            pltpu.sync_copy(w_hbm.at[row], wb)

            @pl.loop(0, K // 16)
            def _xform(j):
                ll = j * 16
                iv = ib[pl.ds(ll, 16)]
                plane = iv >> 16
                rel = plane - lo
                valid = (rel >= 0) & (rel < REG_P)
                tgt = jnp.where(valid, (rel << 16) + (iv & 0xFFFF), DUMP)
                ib[pl.ds(ll, 16)] = tgt

            pltpu.sync_copy(wb, acc.at[ib], add=True)

        plsc.subcore_barrier()

        @pl.loop(0, PLANE_W // K)
        def _wb(cc):
            pltpu.sync_copy(acc.at[pl.ds(s * PLANE_W + cc * K, K)], stage)

            @pl.loop(0, K // 16)
            def _clip(v):
                x = stage[pl.ds(v * 16, 16)]
                stage[pl.ds(v * 16, 16)] = jnp.minimum(
                    jnp.maximum(x, 0.0), 1.0
                )

            pltpu.sync_copy(
                stage, out_hbm.at[pl.ds((lo + s) * PLANE_W + cc * K, K)]
            )

        plsc.subcore_barrier()


@functools.cache
def _sc_scatter():
    return pl.kernel(
        _sc_scatter_body,
        mesh=plsc.VectorSubcoreMesh(core_axis_name="c", subcore_axis_name="s"),
        out_type=jax.ShapeDtypeStruct((RZ * RY * RX,), jnp.float32),
        scratch_types=[
            pltpu.VMEM_SHARED((ACC_N,), jnp.float32),
            pltpu.VMEM((K,), jnp.int32),
            pltpu.VMEM((K,), jnp.float32),
            pltpu.VMEM((K,), jnp.float32),
        ],
    )


@jax.jit
def kernel(vertices, faces):
    faces_i = faces.astype(jnp.int32)
    fv = vertices[faces_i]
    v0t = fv[:, 0, :].T
    v1t = fv[:, 1, :].T
    v2t = fv[:, 2, :].T
    idx56, w56 = _make_pairs(v0t, v1t, v2t)
    idxf = idx56.reshape(-1)
    wf = w56.reshape(-1)
    pad = NP_PAD - NP
    idxf = jnp.concatenate([idxf, jnp.zeros((pad,), jnp.int32)])
    wf = jnp.concatenate([wf, jnp.zeros((pad,), jnp.float32)])
    idx3 = idxf.reshape(NCH_TOT, K)
    w3 = wf.reshape(NCH_TOT, K)
    zeros = jnp.zeros((ACC_N,), jnp.float32)
    occ = _sc_scatter()(idx3, w3, zeros)
    return occ.reshape(RZ, RY, RX)


# speedup vs baseline: 1.0016x; 1.0016x over previous
import functools
import jax
import jax.numpy as jnp
from jax import lax
from jax.experimental import pallas as pl
from jax.experimental.pallas import tpu as pltpu
from jax.experimental.pallas import tpu_sc as plsc

RX = RY = RZ = 256
HN = 2.0
F = 200000
BLK = 1024
GRID = (F + BLK - 1) // BLK

NC, NS = 2, 16
K = 16384
KROWS = K // 128
NP = 56 * F
NCH = -(-NP // (NS * K))
NCH_TOT = NS * NCH
NP_PAD = NCH_TOT * K

REG_P = 16
PLANE_W = RY * RX
ACC_N = REG_P * PLANE_W + 8
DUMP = REG_P * PLANE_W
SWEEPS = (RZ // NC) // REG_P


def _prep_body(v0r, v1r, v2r, idxo, wo):
    v0 = v0r[...]
    v1 = v1r[...]
    v2 = v2r[...]
    pts = (
        v0,
        v1,
        v2,
        (v0 + v1 + v2) / 3.0,
        (v0 + v1) * 0.5,
        (v1 + v2) * 0.5,
        (v0 + v2) * 0.5,
    )
    res = jnp.float32(RX)
    for p_i, p in enumerate(pts):
        g = (p / HN + 0.5) * res - 0.5
        g0f = jnp.floor(g)
        fr = g - g0f
        g0 = g0f.astype(jnp.int32)
        for dx in (0, 1):
            wx = fr[0:1, :] if dx == 1 else (1.0 - fr[0:1, :])
            ix = jnp.clip(g0[0:1, :] + dx, 0, RX - 1)
            for dy in (0, 1):
                wy = fr[1:2, :] if dy == 1 else (1.0 - fr[1:2, :])
                iy = jnp.clip(g0[1:2, :] + dy, 0, RY - 1)
                for dz in (0, 1):
                    wz = fr[2:3, :] if dz == 1 else (1.0 - fr[2:3, :])
                    iz = jnp.clip(g0[2:3, :] + dz, 0, RZ - 1)
                    row = p_i * 8 + dx * 4 + dy * 2 + dz
                    flat = (iz * RY + iy) * RX + ix
                    idxo[row : row + 1, :] = flat
                    wo[row : row + 1, :] = wx * wy * wz


def _make_pairs(v0t, v1t, v2t):
    return pl.pallas_call(
        _prep_body,
        grid=(GRID,),
        in_specs=[pl.BlockSpec((3, BLK), lambda i: (0, i))] * 3,
        out_specs=[
            pl.BlockSpec((56, BLK), lambda i: (0, i)),
            pl.BlockSpec((56, BLK), lambda i: (0, i)),
        ],
        out_shape=[
            jax.ShapeDtypeStruct((56, F), jnp.int32),
            jax.ShapeDtypeStruct((56, F), jnp.float32),
        ],
    )(v0t, v1t, v2t)


def _sc_scatter_body(idx_hbm, w_hbm, z_hbm, out_hbm, acc, ib, wb, stage, sem):
    c = lax.axis_index("c")
    s = lax.axis_index("s")

    for r in range(SWEEPS):
        lo = c * (RZ // NC) + r * REG_P

        @pl.when(s == 0)
        def _():
            pltpu.sync_copy(z_hbm, acc)

        plsc.subcore_barrier()

        @pl.loop(0, NCH)
        def _chunk(ci):
            row = s * NCH + ci
            cp = pltpu.async_copy(w_hbm.at[row], wb, sem)
            pltpu.sync_copy(idx_hbm.at[row], ib)

            @pl.loop(0, K // 16, unroll=8)
            def _xform(j):
                ll = j * 16
                iv = ib[pl.ds(ll, 16)]
                plane = iv >> 16
                rel = plane - lo
                valid = (rel >= 0) & (rel < REG_P)
                tgt = jnp.where(valid, (rel << 16) + (iv & 0xFFFF), DUMP)
                ib[pl.ds(ll, 16)] = tgt

            cp.wait()
            pltpu.sync_copy(wb, acc.at[ib], add=True)

        plsc.subcore_barrier()

        @pl.loop(0, PLANE_W // K)
        def _wb(cc):
            pltpu.sync_copy(acc.at[pl.ds(s * PLANE_W + cc * K, K)], stage)

            @pl.loop(0, K // 16, unroll=8)
            def _clip(v):
                x = stage[pl.ds(v * 16, 16)]
                stage[pl.ds(v * 16, 16)] = jnp.minimum(
                    jnp.maximum(x, 0.0), 1.0
                )

            pltpu.sync_copy(
                stage, out_hbm.at[pl.ds((lo + s) * PLANE_W + cc * K, K)]
            )

        plsc.subcore_barrier()


@functools.cache
def _sc_scatter():
    return pl.kernel(
        _sc_scatter_body,
        mesh=plsc.VectorSubcoreMesh(core_axis_name="c", subcore_axis_name="s"),
        out_type=jax.ShapeDtypeStruct((RZ * RY * RX,), jnp.float32),
        scratch_types=[
            pltpu.VMEM_SHARED((ACC_N,), jnp.float32),
            pltpu.VMEM((K,), jnp.int32),
            pltpu.VMEM((K,), jnp.float32),
            pltpu.VMEM((K,), jnp.float32),
            pltpu.SemaphoreType.DMA,
        ],
    )


@jax.jit
def kernel(vertices, faces):
    faces_i = faces.astype(jnp.int32)
    fv = vertices[faces_i]
    v0t = fv[:, 0, :].T
    v1t = fv[:, 1, :].T
    v2t = fv[:, 2, :].T
    idx56, w56 = _make_pairs(v0t, v1t, v2t)
    idxf = idx56.reshape(-1)
    wf = w56.reshape(-1)
    pad = NP_PAD - NP
    idxf = jnp.concatenate([idxf, jnp.zeros((pad,), jnp.int32)])
    wf = jnp.concatenate([wf, jnp.zeros((pad,), jnp.float32)])
    idx3 = idxf.reshape(NCH_TOT, K)
    w3 = wf.reshape(NCH_TOT, K)
    zeros = jnp.zeros((ACC_N,), jnp.float32)
    occ = _sc_scatter()(idx3, w3, zeros)
    return occ.reshape(RZ, RY, RX)
